# baseline (speedup 1.0000x reference)
import functools
import math

import jax
import jax.numpy as jnp
from jax.experimental import pallas as pl
from jax.experimental.pallas import tpu as pltpu


def _pool_mm_kernel(x_ref, m_ref, o_ref, *, inv_count):
    acc = jax.lax.dot_general(
        x_ref[...], m_ref[...],
        dimension_numbers=(((1,), (0,)), ((), ())),
        preferred_element_type=jnp.float32,
    )
    o_ref[...] = (acc * inv_count).astype(o_ref.dtype)


def kernel(x):
    N, C, H, W = x.shape
    K = H * W
    if x.size == 0:
        return jnp.zeros((N, C), dtype=x.dtype)

    F = C * K
    g = (K * 128) // math.gcd(K, 128)
    cg = g // K
    assert F % g == 0, "channel count must tile the lane group"
    G = F // g

    x2 = x.reshape(N, F)
    ii = jax.lax.broadcasted_iota(jnp.int32, (g, cg), 0)
    jj = jax.lax.broadcasted_iota(jnp.int32, (g, cg), 1)
    mask = (ii // K == jj).astype(x.dtype)

    bn = min(N, 128)
    return pl.pallas_call(
        functools.partial(_pool_mm_kernel, inv_count=1.0 / float(K)),
        out_shape=jax.ShapeDtypeStruct((N, C), x.dtype),
        grid=(pl.cdiv(N, bn), G),
        in_specs=[
            pl.BlockSpec((bn, g), lambda i, t: (i, t)),
            pl.BlockSpec((g, cg), lambda i, t: (0, 0)),
        ],
        out_specs=pl.BlockSpec((bn, cg), lambda i, t: (i, t)),
        compiler_params=pltpu.CompilerParams(
            dimension_semantics=("parallel", "parallel"),
        ),
    )(x2, mask)


# speedup vs baseline: 9.2388x; 9.2388x over previous
import functools

import jax
import jax.numpy as jnp
from jax.experimental import pallas as pl
from jax.experimental.pallas import tpu as pltpu


def _plane_sum_kernel(x_ref, o_ref, *, inv_count):
    s = jnp.sum(x_ref[...].astype(jnp.float32), axis=0)
    o_ref[...] = (s * inv_count).astype(o_ref.dtype)


def kernel(x):
    N, C, H, W = x.shape
    K = H * W
    if x.size == 0:
        return jnp.zeros((N, C), dtype=x.dtype)

    xp = x.transpose(2, 3, 0, 1).reshape(K, N, C)

    bn = 32 if N % 32 == 0 else N
    return pl.pallas_call(
        functools.partial(_plane_sum_kernel, inv_count=1.0 / float(K)),
        out_shape=jax.ShapeDtypeStruct((N, C), x.dtype),
        grid=(N // bn,),
        in_specs=[pl.BlockSpec((K, bn, C), lambda i: (0, i, 0))],
        out_specs=pl.BlockSpec((bn, C), lambda i: (i, 0)),
        compiler_params=pltpu.CompilerParams(
            dimension_semantics=("parallel",),
        ),
    )(xp)
